# baseline (speedup 1.0000x reference)
import math

import jax
import jax.numpy as jnp
from jax.experimental import pallas as pl
from jax.experimental.pallas import tpu as pltpu

K = 2
N = 10000
NFEAT = 128
NH = 64
NCLASS = 40
NLAYERS = 2
LAMDA = 0.5
ALPHA = 0.1

BM = 400


def _fc_body(x_ref, wt_ref, b_ref, o_ref):
    acc = jnp.dot(x_ref[0], wt_ref[0], preferred_element_type=jnp.float32)
    o_ref[0] = jnp.maximum(acc + b_ref[0], 0.0)


def _fc_stage(x, fc_Wt, fc_b3):
    return pl.pallas_call(
        _fc_body,
        grid=(K,),
        in_specs=[
            pl.BlockSpec((1, N, NFEAT), lambda k: (k, 0, 0)),
            pl.BlockSpec((1, NFEAT, NH), lambda k: (k, 0, 0)),
            pl.BlockSpec((1, 1, NH), lambda k: (k, 0, 0)),
        ],
        out_specs=pl.BlockSpec((1, N, NH), lambda k: (k, 0, 0)),
        out_shape=jax.ShapeDtypeStruct((K, N, NH), jnp.float32),
        compiler_params=pltpu.CompilerParams(
            dimension_semantics=("arbitrary",)),
    )(x, fc_Wt, fc_b3)


def _ortho_body(w_ref, m_ref):
    i = pl.program_id(0)
    W = w_ref[0]
    wtw = jax.lax.dot_general(W, W, (((0,), (0,)), ((), ())),
                              preferred_element_type=jnp.float32)
    rows = jax.lax.broadcasted_iota(jnp.int32, (NH, 1), 0)
    lanes = jax.lax.broadcasted_iota(jnp.int32, (1, NH), 1)
    eye = (rows == lanes).astype(jnp.float32)
    A0 = wtw + 1e-4 * eye

    def chol_step(k, AL):
        A, L = AL
        colm = (lanes == k)
        rowm = (rows == k)
        akk = jnp.sum(jnp.where(rowm & colm, A, 0.0))
        inv = jax.lax.rsqrt(akk)
        colv = jnp.sum(jnp.where(colm, A, 0.0), axis=1, keepdims=True)
        rowv = jnp.sum(jnp.where(rowm, A, 0.0), axis=0, keepdims=True)
        lcol = jnp.where(rows >= k, colv * inv, 0.0)
        lrow = jnp.where(lanes >= k, rowv * inv, 0.0)
        A = A - lcol * lrow
        L = L + lcol * colm.astype(jnp.float32)
        return A, L

    zero = jnp.zeros((NH, NH), jnp.float32)
    _, L = jax.lax.fori_loop(0, NH, chol_step, (A0, zero))

    def solve_step(j, X):
        colm = (lanes == j)
        rowm = (rows == j)
        lrow_j = jnp.sum(jnp.where(rowm, L, 0.0), axis=0, keepdims=True)
        ljj = jnp.sum(jnp.where(rowm & colm, L, 0.0))
        acc = jnp.sum(X * lrow_j, axis=1, keepdims=True)
        wcol = jnp.sum(jnp.where(colm, W, 0.0), axis=1, keepdims=True)
        xcol = (wcol - acc) / ljj
        return X + xcol * colm.astype(jnp.float32)

    X = jax.lax.fori_loop(0, NH, solve_step, zero)

    t0 = math.log(LAMDA / 1.0 + 1.0)
    t1 = math.log(LAMDA / 2.0 + 1.0)
    theta = jnp.where(i == 0, jnp.float32(t0), jnp.float32(t1))
    m_ref[0] = theta * X + (1.0 - theta) * eye


def _ortho_stage(conv_W):
    return pl.pallas_call(
        _ortho_body,
        grid=(NLAYERS,),
        in_specs=[pl.BlockSpec((1, NH, NH), lambda i: (i, 0, 0))],
        out_specs=pl.BlockSpec((1, NH, NH), lambda i: (i, 0, 0)),
        out_shape=jax.ShapeDtypeStruct((NLAYERS, NH, NH), jnp.float32),
        compiler_params=pltpu.CompilerParams(
            dimension_semantics=("arbitrary",)),
    )(conv_W)


def _layer_body_plain(adj_ref, h_ref, h0_ref, m_ref, o_ref):
    hi = jnp.dot(adj_ref[0], h_ref[...], preferred_element_type=jnp.float32)
    support = (1.0 - ALPHA) * hi + ALPHA * h0_ref[...]
    z = jnp.dot(support, m_ref[...], preferred_element_type=jnp.float32)
    o_ref[...] = jnp.maximum(jnp.tanh(z), 0.0)


def _layer_body_mix(mix_out_first, adj_ref, h_ref, h0_ref, m_ref, other_ref,
                    w_ref, o_ref, mix_ref):
    hi = jnp.dot(adj_ref[0], h_ref[...], preferred_element_type=jnp.float32)
    support = (1.0 - ALPHA) * hi + ALPHA * h0_ref[...]
    z = jnp.dot(support, m_ref[...], preferred_element_type=jnp.float32)
    out = jnp.maximum(jnp.tanh(z), 0.0)
    o_ref[...] = out
    w = w_ref[0, 0]
    if mix_out_first:
        mix_ref[...] = w * out + (1.0 - w) * other_ref[...]
    else:
        mix_ref[...] = w * other_ref[...] + (1.0 - w) * out


def _layer_stage(adj, k, H, h0, M, other=None, w2d=None, mix_out_first=False):
    grid = (N // BM,)
    adj_spec = pl.BlockSpec((1, BM, N), lambda i: (k, i, 0))
    h_spec = pl.BlockSpec((N, NH), lambda i: (0, 0))
    tile_spec = pl.BlockSpec((BM, NH), lambda i: (i, 0))
    m_spec = pl.BlockSpec((NH, NH), lambda i: (0, 0))
    tile_shape = jax.ShapeDtypeStruct((N, NH), jnp.float32)
    params = pltpu.CompilerParams(dimension_semantics=("arbitrary",),
                                  vmem_limit_bytes=100 * 1024 * 1024)
    if other is None:
        return pl.pallas_call(
            _layer_body_plain,
            grid=grid,
            in_specs=[adj_spec, h_spec, tile_spec, m_spec],
            out_specs=tile_spec,
            out_shape=tile_shape,
            compiler_params=params,
        )(adj, H, h0, M)
    import functools
    body = functools.partial(_layer_body_mix, mix_out_first)
    return pl.pallas_call(
        body,
        grid=grid,
        in_specs=[adj_spec, h_spec, tile_spec, m_spec, tile_spec,
                  pl.BlockSpec(memory_space=pltpu.SMEM)],
        out_specs=(tile_spec, tile_spec),
        out_shape=(tile_shape, tile_shape),
        compiler_params=params,
    )(adj, H, h0, M, other, w2d)


def _final_body(o00_ref, o01_ref, o10_ref, o11_ref, wt_ref, b_ref,
                fin_ref, mean_ref, logs_ref):
    s0 = o00_ref[...] + o01_ref[...]
    s1 = o10_ref[...] + o11_ref[...]
    wt = wt_ref[...]
    b = b_ref[...]
    l0 = jnp.dot(s0, wt, preferred_element_type=jnp.float32) + b
    l1 = jnp.dot(s1, wt, preferred_element_type=jnp.float32) + b

    def logsoftmax(z):
        m = jnp.max(z, axis=1, keepdims=True)
        e = z - m
        return e - jnp.log(jnp.sum(jnp.exp(e), axis=1, keepdims=True))

    ls0 = logsoftmax(l0)
    ls1 = logsoftmax(l1)
    fin_ref[...] = logsoftmax(l0 + l1)
    mean_ref[...] = 0.5 * (ls0 + ls1)
    logs_ref[0] = ls0
    logs_ref[1] = ls1


def _final_stage(o00, o01, o10, o11, fco_Wt, fco_b2):
    tile = pl.BlockSpec((N, NH), lambda: (0, 0))
    return pl.pallas_call(
        _final_body,
        in_specs=[tile, tile, tile, tile,
                  pl.BlockSpec((NH, NCLASS), lambda: (0, 0)),
                  pl.BlockSpec((1, NCLASS), lambda: (0, 0))],
        out_specs=(pl.BlockSpec((N, NCLASS), lambda: (0, 0)),
                   pl.BlockSpec((N, NCLASS), lambda: (0, 0)),
                   pl.BlockSpec((K, N, NCLASS), lambda: (0, 0, 0))),
        out_shape=(jax.ShapeDtypeStruct((N, NCLASS), jnp.float32),
                   jax.ShapeDtypeStruct((N, NCLASS), jnp.float32),
                   jax.ShapeDtypeStruct((K, N, NCLASS), jnp.float32)),
        compiler_params=pltpu.CompilerParams(
            vmem_limit_bytes=100 * 1024 * 1024),
    )(o00, o01, o10, o11, fco_Wt, fco_b2)


def kernel(x, adj, conv_W, fc_W, fc_b, fco_W, fco_b, w):
    fc_Wt = jnp.swapaxes(fc_W, 1, 2)
    fc_b3 = fc_b[:, None, :]
    fco_Wt = fco_W.T
    fco_b2 = fco_b[None, :]
    w2d = w.reshape(1, 1)

    fc = _fc_stage(x, fc_Wt, fc_b3)
    M = _ortho_stage(conv_W)
    h00 = fc[0]
    h01 = fc[1]
    M0 = M[0]
    M1 = M[1]

    out00, mix10 = _layer_stage(adj, 0, h00, h00, M0,
                                other=h01, w2d=w2d, mix_out_first=False)
    out01 = _layer_stage(adj, 0, out00, h00, M1)
    out10, mix11 = _layer_stage(adj, 1, mix10, h01, M0,
                                other=out01, w2d=w2d, mix_out_first=True)
    out11 = _layer_stage(adj, 1, mix11, h01, M1)

    fin, mean, logs = _final_stage(out00, out01, out10, out11,
                                   fco_Wt, fco_b2)
    return fin, mean, logs, w
